# baseline (speedup 1.0000x reference)
import functools

import jax
import jax.numpy as jnp
from jax import lax
from jax.experimental import pallas as pl
from jax.experimental.pallas import tpu as pltpu
from jax.experimental.pallas import tpu_sc as plsc


def _mm_body(ta_ref, tb_ref, w0_ref, w1_ref, o_ref):
    ta = ta_ref[...].astype(jnp.bfloat16)
    tb = tb_ref[...].astype(jnp.bfloat16)
    w0 = w0_ref[...]
    w1 = w1_ref[...]
    a0 = jnp.concatenate(
        [jnp.dot(ta, w0, preferred_element_type=jnp.float32),
         jnp.dot(tb, w0, preferred_element_type=jnp.float32)], axis=1)
    a1 = jnp.concatenate(
        [jnp.dot(ta, w1, preferred_element_type=jnp.float32),
         jnp.dot(tb, w1, preferred_element_type=jnp.float32)], axis=1)
    u0 = lax.bitcast_convert_type(a0, jnp.uint32)
    u1 = lax.bitcast_convert_type(a1, jnp.uint32)
    half = jnp.uint32(0x8000)
    hi_mask = jnp.uint32(0xFFFF0000)
    w = ((u1 + half) & hi_mask) | ((u0 + half) >> 16)
    o_ref[...] = lax.bitcast_convert_type(w, jnp.float32)


def _make_P(table, w_c0, w_c1, v_blk):
    V, E = table.shape
    half_blocks = (V // 2) // v_blk
    return pl.pallas_call(
        _mm_body,
        grid=(half_blocks,),
        in_specs=[
            pl.BlockSpec((v_blk, E), lambda i: (i, 0)),
            pl.BlockSpec((v_blk, E), lambda i, hb=half_blocks: (i + hb, 0)),
            pl.BlockSpec((E, 64), lambda i: (0, 0)),
            pl.BlockSpec((E, 64), lambda i: (0, 0)),
        ],
        out_specs=pl.BlockSpec((v_blk, 128), lambda i: (i, 0)),
        out_shape=jax.ShapeDtypeStruct((V // 2, 128), jnp.float32),
    )(table, table, w_c0, w_c1)


def _ls_body(z_ref, b_ref, o_ref):
    z = z_ref[...] + b_ref[0:1, :]
    m = jnp.max(z, axis=-1, keepdims=True)
    e = jnp.exp(z - m)
    s = jnp.sum(e, axis=-1, keepdims=True)
    o_ref[...] = (z - m) - jnp.log(s)


def _log_softmax(z, b8):
    B, C = z.shape
    return pl.pallas_call(
        _ls_body,
        in_specs=[
            pl.BlockSpec((B, C), lambda: (0, 0)),
            pl.BlockSpec(b8.shape, lambda: (0, 0)),
        ],
        out_specs=pl.BlockSpec((B, C), lambda: (0, 0)),
        out_shape=jax.ShapeDtypeStruct((B, C), jnp.float32),
    )(z, b8)


_NC, _NS, _LN = 2, 16, 16
_NW = _NC * _NS


def _make_sc_gather(B, L, C, V):
    assert C == 2
    _VHALF = V // 2
    rows_w = B // _NW
    n_idx = rows_w * L
    n_jrows = n_idx // 128
    n_chunks = n_idx // _LN
    n_sub = rows_w // _LN

    mesh = plsc.VectorSubcoreMesh(core_axis_name="c", subcore_axis_name="s")

    @functools.partial(
        pl.kernel,
        mesh=mesh,
        compiler_params=pltpu.CompilerParams(needs_layout_passes=False),
        out_type=jax.ShapeDtypeStruct((B * C,), jnp.float32),
        scratch_types=[
            pltpu.VMEM((n_idx,), jnp.int32),
            pltpu.VMEM((n_jrows, 128), jnp.int32),
            pltpu.VMEM((n_idx,), jnp.float32),
            pltpu.VMEM((2 * rows_w,), jnp.float32),
            pltpu.SemaphoreType.DMA,
        ],
    )
    def sc_gather(pf_hbm, inp_hbm, out_hbm, inp_v, idx_v, d_v, out_v, sem):
        cid = lax.axis_index("c")
        sid = lax.axis_index("s")
        wid = sid * _NC + cid

        iota = lax.iota(jnp.int32, _LN)
        l_div = jnp.full((_LN,), L, jnp.int32)
        zf = jnp.full((_LN,), 0.0, jnp.float32)
        sh16 = jnp.full((_LN,), 16, jnp.uint32)
        hi_mask = jnp.full((_LN,), 0xFFFF0000, jnp.uint32)
        vhalf = jnp.full((_LN,), _VHALF, jnp.int32)
        hi_off = jnp.full((_LN,), 64 - _VHALF * 128, jnp.int32)
        zero16 = jnp.full((_LN,), 0, jnp.int32)

        pltpu.sync_copy(inp_hbm.at[pl.ds(wid * n_idx, n_idx)], inp_v)

        for k in range(n_chunks):
            v = inp_v[pl.ds(k * _LN, _LN)]
            lpos = lax.rem(iota + (k * _LN), l_div)
            sel = jnp.where(v < vhalf, zero16, hi_off)
            idx_v[k // 8, pl.ds((k % 8) * _LN, _LN)] = v * 128 + lpos + sel

        cps = [
            pltpu.async_copy(
                pf_hbm.at[idx_v.at[j]], d_v.at[pl.ds(j * 128, 128)], sem)
            for j in range(n_jrows)
        ]
        for cp in cps:
            cp.wait()

        for gi in range(n_sub):
            base = gi * _LN * L

            def _red(l, accs, base=base):
                a0, a1 = accs
                ridx = base + iota * L + l
                u = plsc.bitcast(plsc.load_gather(d_v, [ridx]), jnp.uint32)
                c0 = plsc.bitcast(lax.shift_left(u, sh16), jnp.float32)
                c1 = plsc.bitcast(u & hi_mask, jnp.float32)
                return (a0 + c0, a1 + c1)

            a0, a1 = lax.fori_loop(0, L, _red, (zf, zf))
            plsc.store_scatter(out_v, [gi * 2 * _LN + iota * 2], a0)
            plsc.store_scatter(out_v, [gi * 2 * _LN + iota * 2 + 1], a1)

        pltpu.sync_copy(out_v, out_hbm.at[pl.ds(wid * 2 * rows_w, 2 * rows_w)])

    return sc_gather


def kernel(input, table, W, b):
    B, L = input.shape
    V, E = table.shape
    C = W.shape[1]

    w_lec = W.reshape(L, E, C)
    w_c0 = jnp.pad(w_lec[:, :, 0].T, ((0, 0), (0, 64 - L))).astype(jnp.bfloat16)
    w_c1 = jnp.pad(w_lec[:, :, 1].T, ((0, 0), (0, 64 - L))).astype(jnp.bfloat16)

    P = _make_P(table, w_c0, w_c1, v_blk=5000)
    pf = P.reshape((V // 2) * 128)

    inp_flat = input.reshape(-1).astype(jnp.int32)
    logits = _make_sc_gather(B, L, C, V)(pf, inp_flat).reshape(B, C)

    b8 = jnp.broadcast_to(b.reshape(1, C).astype(jnp.float32), (8, C))
    return _log_softmax(logits, b8)


# speedup vs baseline: 1.0219x; 1.0219x over previous
import functools

import jax
import jax.numpy as jnp
from jax import lax
from jax.experimental import pallas as pl
from jax.experimental.pallas import tpu as pltpu
from jax.experimental.pallas import tpu_sc as plsc


def _mm_body(ta_ref, tb_ref, w0_ref, w1_ref, o_ref):
    ta = ta_ref[...].astype(jnp.bfloat16)
    tb = tb_ref[...].astype(jnp.bfloat16)
    w0 = w0_ref[...]
    w1 = w1_ref[...]
    a0 = jnp.concatenate(
        [jnp.dot(ta, w0, preferred_element_type=jnp.float32),
         jnp.dot(tb, w0, preferred_element_type=jnp.float32)], axis=1)
    a1 = jnp.concatenate(
        [jnp.dot(ta, w1, preferred_element_type=jnp.float32),
         jnp.dot(tb, w1, preferred_element_type=jnp.float32)], axis=1)
    u0 = lax.bitcast_convert_type(a0, jnp.uint32)
    u1 = lax.bitcast_convert_type(a1, jnp.uint32)
    half = jnp.uint32(0x8000)
    hi_mask = jnp.uint32(0xFFFF0000)
    w = ((u1 + half) & hi_mask) | ((u0 + half) >> 16)
    o_ref[...] = lax.bitcast_convert_type(w, jnp.float32)


def _make_P(table, w_c0, w_c1, v_blk):
    V, E = table.shape
    half_blocks = (V // 2) // v_blk
    return pl.pallas_call(
        _mm_body,
        grid=(half_blocks,),
        in_specs=[
            pl.BlockSpec((v_blk, E), lambda i: (i, 0)),
            pl.BlockSpec((v_blk, E), lambda i, hb=half_blocks: (i + hb, 0)),
            pl.BlockSpec((E, 64), lambda i: (0, 0)),
            pl.BlockSpec((E, 64), lambda i: (0, 0)),
        ],
        out_specs=pl.BlockSpec((v_blk, 128), lambda i: (i, 0)),
        out_shape=jax.ShapeDtypeStruct((V // 2, 128), jnp.float32),
    )(table, table, w_c0, w_c1)


_NC, _NS, _LN = 2, 16, 16
_NW = _NC * _NS


def _make_sc_gather(B, L, C, V):
    assert C == 2
    _VHALF = V // 2
    rows_w = B // _NW
    n_idx = rows_w * L
    n_jrows = n_idx // 128
    n_chunks = n_idx // _LN
    n_sub = rows_w // _LN

    mesh = plsc.VectorSubcoreMesh(core_axis_name="c", subcore_axis_name="s")

    @functools.partial(
        pl.kernel,
        mesh=mesh,
        compiler_params=pltpu.CompilerParams(needs_layout_passes=False),
        out_type=jax.ShapeDtypeStruct((B * C,), jnp.float32),
        scratch_types=[
            pltpu.VMEM((n_idx,), jnp.int32),
            pltpu.VMEM((n_jrows, 128), jnp.int32),
            pltpu.VMEM((n_idx,), jnp.float32),
            pltpu.VMEM((2 * rows_w,), jnp.float32),
            pltpu.VMEM((2 * _LN,), jnp.float32),
            pltpu.SemaphoreType.DMA,
        ],
    )
    def sc_gather(pf_hbm, inp_hbm, bvec_hbm, out_hbm, inp_v, idx_v, d_v, out_v,
                  b_v, sem):
        cid = lax.axis_index("c")
        sid = lax.axis_index("s")
        wid = sid * _NC + cid

        iota = lax.iota(jnp.int32, _LN)
        l_div = jnp.full((_LN,), L, jnp.int32)
        zf = jnp.full((_LN,), 0.0, jnp.float32)
        sh16 = jnp.full((_LN,), 16, jnp.uint32)
        hi_mask = jnp.full((_LN,), 0xFFFF0000, jnp.uint32)
        vhalf = jnp.full((_LN,), _VHALF, jnp.int32)
        hi_off = jnp.full((_LN,), 64 - _VHALF * 128, jnp.int32)
        zero16 = jnp.full((_LN,), 0, jnp.int32)

        pltpu.sync_copy(inp_hbm.at[pl.ds(wid * n_idx, n_idx)], inp_v)
        pltpu.sync_copy(bvec_hbm, b_v)

        for k in range(n_chunks):
            v = inp_v[pl.ds(k * _LN, _LN)]
            lpos = lax.rem(iota + (k * _LN), l_div)
            sel = jnp.where(v < vhalf, zero16, hi_off)
            idx_v[k // 8, pl.ds((k % 8) * _LN, _LN)] = v * 128 + lpos + sel

        cps = [
            pltpu.async_copy(
                pf_hbm.at[idx_v.at[j]], d_v.at[pl.ds(j * 128, 128)], sem)
            for j in range(n_jrows)
        ]
        for cp in cps:
            cp.wait()

        for gi in range(n_sub):
            base = gi * _LN * L

            def _red(l, accs, base=base):
                a0, a1 = accs
                ridx = base + iota * L + l
                u = plsc.bitcast(plsc.load_gather(d_v, [ridx]), jnp.uint32)
                c0 = plsc.bitcast(lax.shift_left(u, sh16), jnp.float32)
                c1 = plsc.bitcast(u & hi_mask, jnp.float32)
                return (a0 + c0, a1 + c1)

            a0, a1 = lax.fori_loop(0, L, _red, (zf, zf))

            z0 = a0 + b_v[pl.ds(0, _LN)]
            z1 = a1 + b_v[pl.ds(_LN, _LN)]
            m = jnp.maximum(z0, z1)
            d0 = z0 - m
            d1 = z1 - m
            s = jnp.exp(d0) + jnp.exp(d1)
            y = (s - 1.0) * 0.6931472
            for _ in range(3):
                y = y - 1.0 + s * jnp.exp(-y)
            plsc.store_scatter(out_v, [gi * 2 * _LN + iota * 2], d0 - y)
            plsc.store_scatter(out_v, [gi * 2 * _LN + iota * 2 + 1], d1 - y)

        pltpu.sync_copy(out_v, out_hbm.at[pl.ds(wid * 2 * rows_w, 2 * rows_w)])

    return sc_gather


def kernel(input, table, W, b):
    B, L = input.shape
    V, E = table.shape
    C = W.shape[1]

    w_lec = W.reshape(L, E, C)
    w_c0 = jnp.pad(w_lec[:, :, 0].T, ((0, 0), (0, 64 - L))).astype(jnp.bfloat16)
    w_c1 = jnp.pad(w_lec[:, :, 1].T, ((0, 0), (0, 64 - L))).astype(jnp.bfloat16)

    P = _make_P(table, w_c0, w_c1, v_blk=5000)
    pf = P.reshape((V // 2) * 128)

    inp_flat = input.reshape(-1).astype(jnp.int32)
    bvec = jnp.repeat(b.astype(jnp.float32), 16)
    out = _make_sc_gather(B, L, C, V)(pf, inp_flat, bvec)
    return out.reshape(B, C)
